# baseline (speedup 1.0000x reference)
import functools

import jax
import jax.numpy as jnp
from jax import lax
from jax.experimental import pallas as pl
from jax.experimental.pallas import tpu as pltpu
from jax.experimental.pallas import tpu_sc as plsc

_D = 2048
_G = 8
_K = 8192
_DG = _D // _G
_BETA = 0.25
_B = 4096

_BM = 512
_KT = 512
_NB = _B // _BM
_NK = _K // _KT

def _argmin_body(h_ref, p_ref, c_ref, idx_ref, z_ref, best_ref, bidx_ref):
    i = pl.program_id(1)
    k = pl.program_id(2)

    @pl.when(k == 0)
    def _init():
        z_ref[...] = lax.dot_general(
            h_ref[...], p_ref[0],
            (((1,), (1,)), ((), ())),
            preferred_element_type=jnp.float32)
        best_ref[...] = jnp.full((_BM, 1), jnp.inf, jnp.float32)
        bidx_ref[...] = jnp.zeros((_BM, 1), jnp.int32)

    @pl.when(k == _K // (2 * _KT))
    def _roundtrip():
        best_ref[...] = best_ref[...].astype(jnp.bfloat16).astype(jnp.float32)

    z = z_ref[...]
    c = c_ref[0]
    zc = lax.dot_general(z, c, (((1,), (1,)), ((), ())),
                         preferred_element_type=jnp.float32)
    zz = jnp.sum(z * z, axis=1, keepdims=True)
    cc = lax.dot_general(jnp.ones((1, _DG), jnp.float32), c * c,
                         (((1,), (1,)), ((), ())),
                         precision=lax.Precision.HIGHEST,
                         preferred_element_type=jnp.float32)
    d2 = (zz - 2.0 * zc) + cc

    m = jnp.min(d2, axis=1, keepdims=True)
    col = lax.broadcasted_iota(jnp.int32, (_BM, _KT), 1) + k * _KT
    tile_idx = jnp.min(jnp.where(d2 == m, col, 2**31 - 1),
                       axis=1, keepdims=True)
    tie = jnp.logical_and(m == best_ref[...], tile_idx < bidx_ref[...])
    upd = jnp.logical_or(m < best_ref[...], tie)
    bidx_ref[...] = jnp.where(upd, tile_idx, bidx_ref[...])
    best_ref[...] = jnp.where(m < best_ref[...], m, best_ref[...])

    @pl.when(k == _NK - 1)
    def _emit():
        idx_ref[0, 0, pl.ds(i * _BM, _BM)] = bidx_ref[:, 0]


_argmin_call = pl.pallas_call(
    _argmin_body,
    grid=(_G, _NB, _NK),
    in_specs=[
        pl.BlockSpec((_BM, _D), lambda g, i, k: (i, 0)),
        pl.BlockSpec((1, _DG, _D), lambda g, i, k: (g, 0, 0)),
        pl.BlockSpec((1, _KT, _DG), lambda g, i, k: (g, k, 0)),
    ],
    out_specs=pl.BlockSpec((1, 1, _B), lambda g, i, k: (g, 0, 0)),
    out_shape=jax.ShapeDtypeStruct((_G, 1, _B), jnp.int32),
    scratch_shapes=[
        pltpu.VMEM((_BM, _DG), jnp.float32),
        pltpu.VMEM((_BM, 1), jnp.float32),
        pltpu.VMEM((_BM, 1), jnp.int32),
    ],
)


_NC = 2
_NS = 16
_NW = _NC * _NS
_ROWS = _B * _G
_PER_W = _ROWS // _NW
_CH = 128
_NCH = _PER_W // _CH


def _gather_body(table_hbm, idx_hbm, out_hbm, idx_v, rows_v, sem):
    wid = lax.axis_index("s") * _NC + lax.axis_index("c")
    base = wid * _PER_W

    def _chunk(c, carry):
        off = base + c * _CH
        pltpu.sync_copy(idx_hbm.at[pl.ds(off, _CH)], idx_v)
        pltpu.async_copy(table_hbm.at[idx_v], rows_v, sem).wait()
        pltpu.sync_copy(rows_v, out_hbm.at[pl.ds(off, _CH)])
        return carry

    lax.fori_loop(0, _NCH, _chunk, 0)


@functools.cache
def _gather_call():
    return pl.kernel(
        _gather_body,
        out_type=jax.ShapeDtypeStruct((_ROWS, _DG), jnp.float32),
        mesh=plsc.VectorSubcoreMesh(core_axis_name="c", subcore_axis_name="s",
                                    num_cores=_NC, num_subcores=_NS),
        scratch_types=[
            pltpu.VMEM((_CH,), jnp.int32),
            pltpu.VMEM((_CH, _DG), jnp.float32),
            pltpu.SemaphoreType.DMA,
        ],
    )


_BL = 512
_NBL = _B // _BL


def _loss_body(q_ref, h_ref, out_ref, acc_ref):
    i = pl.program_id(0)

    @pl.when(i == 0)
    def _init():
        acc_ref[0, 0] = 0.0

    diff = q_ref[...] - h_ref[...]
    acc_ref[0, 0] += jnp.sum(diff * diff)

    @pl.when(i == _NBL - 1)
    def _emit():
        mean = acc_ref[0, 0] / (_B * _D)
        out_ref[0, 0] = mean + _BETA * mean


_loss_call = pl.pallas_call(
    _loss_body,
    grid=(_NBL,),
    in_specs=[
        pl.BlockSpec((_BL, _D), lambda i: (i, 0)),
        pl.BlockSpec((_BL, _D), lambda i: (i, 0)),
    ],
    out_specs=pl.BlockSpec(memory_space=pltpu.SMEM),
    out_shape=jax.ShapeDtypeStruct((1, 1), jnp.float32),
    scratch_shapes=[pltpu.SMEM((1, 1), jnp.float32)],
)


def kernel(h, projections, codebooks):
    idx3 = _argmin_call(h, projections, codebooks)
    offs = (jnp.arange(_G, dtype=jnp.int32) * _K)[:, None]
    flat_idx = (idx3[:, 0, :] + offs).T.reshape(-1)
    table = codebooks.reshape(_G * _K, _DG)
    quant = _gather_call()(table, flat_idx).reshape(_B, _D)
    loss = _loss_call(quant, h)[0, 0]
    return quant, loss


# speedup vs baseline: 2.4340x; 2.4340x over previous
import functools

import jax
import jax.numpy as jnp
from jax import lax
from jax.experimental import pallas as pl
from jax.experimental.pallas import tpu as pltpu
from jax.experimental.pallas import tpu_sc as plsc

_D = 2048
_G = 8
_K = 8192
_DG = _D // _G
_BETA = 0.25
_B = 4096

_BM = 512
_KT = 2048
_NB = _B // _BM
_NK = _K // _KT
_HALF = _NK // 2
_JG = _KT // 128


def _cnorm_body(c_ref, out_ref):
    c = c_ref[0]
    out_ref[0, 0, :] = jnp.sum(c * c, axis=1)


_cnorm_call = pl.pallas_call(
    _cnorm_body,
    grid=(_G,),
    in_specs=[pl.BlockSpec((1, _K, _DG), lambda g: (g, 0, 0))],
    out_specs=pl.BlockSpec((1, 1, _K), lambda g: (g, 0, 0)),
    out_shape=jax.ShapeDtypeStruct((_G, 1, _K), jnp.float32),
)


def _argmin_body(h_ref, p_ref, c_ref, cc_ref, idx_ref,
                 z2_ref, zz_ref, lval_ref, lgrp_ref, best_ref, bidx_ref):
    i = pl.program_id(1)
    k = pl.program_id(2)

    @pl.when(k == 0)
    def _proj():
        z = lax.dot_general(h_ref[...], p_ref[0], (((1,), (1,)), ((), ())),
                            preferred_element_type=jnp.float32)
        z2_ref[...] = z + z
        zz_ref[...] = jnp.sum(z * z, axis=1, keepdims=True)

    @pl.when(k % _HALF == 0)
    def _chunk_init():
        lval_ref[...] = jnp.full((_BM, 128), jnp.inf, jnp.float32)
        lgrp_ref[...] = jnp.zeros((_BM, 128), jnp.int32)

    zc2 = lax.dot_general(z2_ref[...], c_ref[0], (((1,), (1,)), ((), ())),
                          preferred_element_type=jnp.float32)
    d2 = (zz_ref[...] - zc2) + cc_ref[0, 0, :]

    lval = lval_ref[...]
    lgrp = lgrp_ref[...]
    for j in range(_JG):
        sl = d2[:, j * 128:(j + 1) * 128]
        upd = sl < lval
        lgrp = jnp.where(upd, k * _JG + j, lgrp)
        lval = jnp.where(upd, sl, lval)
    lval_ref[...] = lval
    lgrp_ref[...] = lgrp

    lane = lax.broadcasted_iota(jnp.int32, (_BM, 128), 1)

    @pl.when(k == _HALF - 1)
    def _chunk0_done():
        v = lval_ref[...]
        m = jnp.min(v, axis=1, keepdims=True)
        cand = jnp.where(v == m, lgrp_ref[...] * 128 + lane, 2**31 - 1)
        bidx_ref[...] = jnp.min(cand, axis=1, keepdims=True)
        best_ref[...] = m.astype(jnp.bfloat16).astype(jnp.float32)

    @pl.when(k == _NK - 1)
    def _emit():
        v = lval_ref[...]
        m = jnp.min(v, axis=1, keepdims=True)
        cand = jnp.where(v == m, lgrp_ref[...] * 128 + lane, 2**31 - 1)
        ridx = jnp.min(cand, axis=1, keepdims=True)
        win = m < best_ref[...]
        idx_ref[0, 0, pl.ds(i * _BM, _BM)] = (
            jnp.where(win, ridx, bidx_ref[...]))[:, 0]


_argmin_call = pl.pallas_call(
    _argmin_body,
    grid=(_G, _NB, _NK),
    in_specs=[
        pl.BlockSpec((_BM, _D), lambda g, i, k: (i, 0)),
        pl.BlockSpec((1, _DG, _D), lambda g, i, k: (g, 0, 0)),
        pl.BlockSpec((1, _KT, _DG), lambda g, i, k: (g, k, 0)),
        pl.BlockSpec((1, 1, _KT), lambda g, i, k: (g, 0, k)),
    ],
    out_specs=pl.BlockSpec((1, 1, _B), lambda g, i, k: (g, 0, 0)),
    out_shape=jax.ShapeDtypeStruct((_G, 1, _B), jnp.int32),
    scratch_shapes=[
        pltpu.VMEM((_BM, _DG), jnp.float32),
        pltpu.VMEM((_BM, 1), jnp.float32),
        pltpu.VMEM((_BM, 128), jnp.float32),
        pltpu.VMEM((_BM, 128), jnp.int32),
        pltpu.VMEM((_BM, 1), jnp.float32),
        pltpu.VMEM((_BM, 1), jnp.int32),
    ],
)


_NC = 2
_NS = 16
_NW = _NC * _NS
_ROWS = _B * _G
_PER_W = _ROWS // _NW
_CH = 128
_NCH = _PER_W // _CH


def _gather_body(table_hbm, idx_hbm, out_hbm, idx_v, rows_v, sem):
    wid = lax.axis_index("s") * _NC + lax.axis_index("c")
    base = wid * _PER_W

    def _chunk(c, carry):
        off = base + c * _CH
        pltpu.sync_copy(idx_hbm.at[pl.ds(off, _CH)], idx_v)
        pltpu.async_copy(table_hbm.at[idx_v], rows_v, sem).wait()
        pltpu.sync_copy(rows_v, out_hbm.at[pl.ds(off, _CH)])
        return carry

    lax.fori_loop(0, _NCH, _chunk, 0)


@functools.cache
def _gather_call():
    return pl.kernel(
        _gather_body,
        out_type=jax.ShapeDtypeStruct((_ROWS, _DG), jnp.float32),
        mesh=plsc.VectorSubcoreMesh(core_axis_name="c", subcore_axis_name="s",
                                    num_cores=_NC, num_subcores=_NS),
        scratch_types=[
            pltpu.VMEM((_CH,), jnp.int32),
            pltpu.VMEM((_CH, _DG), jnp.float32),
            pltpu.SemaphoreType.DMA,
        ],
    )


_BL = 512
_NBL = _B // _BL


def _loss_body(q_ref, h_ref, out_ref, acc_ref):
    i = pl.program_id(0)

    @pl.when(i == 0)
    def _init():
        acc_ref[0, 0] = 0.0

    diff = q_ref[...] - h_ref[...]
    acc_ref[0, 0] += jnp.sum(diff * diff)

    @pl.when(i == _NBL - 1)
    def _emit():
        mean = acc_ref[0, 0] / (_B * _D)
        out_ref[0, 0] = mean + _BETA * mean


_loss_call = pl.pallas_call(
    _loss_body,
    grid=(_NBL,),
    in_specs=[
        pl.BlockSpec((_BL, _D), lambda i: (i, 0)),
        pl.BlockSpec((_BL, _D), lambda i: (i, 0)),
    ],
    out_specs=pl.BlockSpec(memory_space=pltpu.SMEM),
    out_shape=jax.ShapeDtypeStruct((1, 1), jnp.float32),
    scratch_shapes=[pltpu.SMEM((1, 1), jnp.float32)],
)


def kernel(h, projections, codebooks):
    cc = _cnorm_call(codebooks)
    idx3 = _argmin_call(h, projections, codebooks, cc)
    offs = (jnp.arange(_G, dtype=jnp.int32) * _K)[:, None]
    flat_idx = (idx3[:, 0, :] + offs).T.reshape(-1)
    table = codebooks.reshape(_G * _K, _DG)
    quant = _gather_call()(table, flat_idx).reshape(_B, _D)
    loss = _loss_call(quant, h)[0, 0]
    return quant, loss


# speedup vs baseline: 2.8723x; 1.1801x over previous
import functools

import jax
import jax.numpy as jnp
from jax import lax
from jax.experimental import pallas as pl
from jax.experimental.pallas import tpu as pltpu
from jax.experimental.pallas import tpu_sc as plsc

_D = 2048
_G = 8
_K = 8192
_DG = _D // _G
_BETA = 0.25
_B = 4096

_BM = 1024
_KT = 2048
_NB = _B // _BM
_NK = _K // _KT
_HALF = _NK // 2
_JG = _KT // 128


def _cnorm_body(c_ref, out_ref):
    c = c_ref[0]
    out_ref[0, 0, :] = jnp.sum(c * c, axis=1)


_cnorm_call = pl.pallas_call(
    _cnorm_body,
    grid=(_G,),
    in_specs=[pl.BlockSpec((1, _K, _DG), lambda g: (g, 0, 0))],
    out_specs=pl.BlockSpec((1, 1, _K), lambda g: (g, 0, 0)),
    out_shape=jax.ShapeDtypeStruct((_G, 1, _K), jnp.float32),
)


def _argmin_body(h_ref, p_ref, c_ref, cc_ref, idx_ref,
                 z2_ref, zz_ref, lval_ref, lgrp_ref, best_ref, bidx_ref):
    i = pl.program_id(1)
    k = pl.program_id(2)

    @pl.when(k == 0)
    def _proj():
        z = lax.dot_general(h_ref[...], p_ref[0], (((1,), (1,)), ((), ())),
                            preferred_element_type=jnp.float32)
        z2_ref[...] = z + z
        zz_ref[...] = jnp.sum(z * z, axis=1, keepdims=True)

    @pl.when(k % _HALF == 0)
    def _chunk_init():
        lval_ref[...] = jnp.full((_BM, 128), jnp.inf, jnp.float32)
        lgrp_ref[...] = jnp.zeros((_BM, 128), jnp.int32)

    zc2 = lax.dot_general(z2_ref[...], c_ref[0], (((1,), (1,)), ((), ())),
                          preferred_element_type=jnp.float32)
    zz = zz_ref[...]
    cc = cc_ref[0, 0, :]

    lval = lval_ref[...]
    lgrp = lgrp_ref[...]
    for j in range(_JG):
        sl = (zz - zc2[:, j * 128:(j + 1) * 128]) + cc[j * 128:(j + 1) * 128]
        upd = sl < lval
        lgrp = jnp.where(upd, k * _JG + j, lgrp)
        lval = jnp.where(upd, sl, lval)
    lval_ref[...] = lval
    lgrp_ref[...] = lgrp

    lane = lax.broadcasted_iota(jnp.int32, (_BM, 128), 1)

    @pl.when(k == _HALF - 1)
    def _chunk0_done():
        v = lval_ref[...]
        m = jnp.min(v, axis=1, keepdims=True)
        cand = jnp.where(v == m, lgrp_ref[...] * 128 + lane, 2**31 - 1)
        bidx_ref[...] = jnp.min(cand, axis=1, keepdims=True)
        best_ref[...] = m.astype(jnp.bfloat16).astype(jnp.float32)

    @pl.when(k == _NK - 1)
    def _emit():
        v = lval_ref[...]
        m = jnp.min(v, axis=1, keepdims=True)
        cand = jnp.where(v == m, lgrp_ref[...] * 128 + lane, 2**31 - 1)
        ridx = jnp.min(cand, axis=1, keepdims=True)
        win = m < best_ref[...]
        idx_ref[0, 0, pl.ds(i * _BM, _BM)] = (
            jnp.where(win, ridx, bidx_ref[...]))[:, 0]


_argmin_call = pl.pallas_call(
    _argmin_body,
    grid=(_G, _NB, _NK),
    in_specs=[
        pl.BlockSpec((_BM, _D), lambda g, i, k: (i, 0)),
        pl.BlockSpec((1, _DG, _D), lambda g, i, k: (g, 0, 0)),
        pl.BlockSpec((1, _KT, _DG), lambda g, i, k: (g, k, 0)),
        pl.BlockSpec((1, 1, _KT), lambda g, i, k: (g, 0, k)),
    ],
    out_specs=pl.BlockSpec((1, 1, _B), lambda g, i, k: (g, 0, 0)),
    out_shape=jax.ShapeDtypeStruct((_G, 1, _B), jnp.int32),
    scratch_shapes=[
        pltpu.VMEM((_BM, _DG), jnp.float32),
        pltpu.VMEM((_BM, 1), jnp.float32),
        pltpu.VMEM((_BM, 128), jnp.float32),
        pltpu.VMEM((_BM, 128), jnp.int32),
        pltpu.VMEM((_BM, 1), jnp.float32),
        pltpu.VMEM((_BM, 1), jnp.int32),
    ],
)


_NC = 2
_NS = 16
_NW = _NC * _NS
_ROWS = _B * _G
_PER_W = _ROWS // _NW
_CH = 128
_NCH = _PER_W // _CH


def _gather_body(table_hbm, idx_hbm, out_hbm, idx_v, rows_v, sem):
    wid = lax.axis_index("s") * _NC + lax.axis_index("c")
    base = wid * _PER_W

    def _chunk(c, carry):
        off = base + c * _CH
        pltpu.sync_copy(idx_hbm.at[pl.ds(off, _CH)], idx_v)
        pltpu.async_copy(table_hbm.at[idx_v], rows_v, sem).wait()
        pltpu.sync_copy(rows_v, out_hbm.at[pl.ds(off, _CH)])
        return carry

    lax.fori_loop(0, _NCH, _chunk, 0)


@functools.cache
def _gather_call():
    return pl.kernel(
        _gather_body,
        out_type=jax.ShapeDtypeStruct((_ROWS, _DG), jnp.float32),
        mesh=plsc.VectorSubcoreMesh(core_axis_name="c", subcore_axis_name="s",
                                    num_cores=_NC, num_subcores=_NS),
        scratch_types=[
            pltpu.VMEM((_CH,), jnp.int32),
            pltpu.VMEM((_CH, _DG), jnp.float32),
            pltpu.SemaphoreType.DMA,
        ],
    )


_BL = 512
_NBL = _B // _BL


def _loss_body(q_ref, h_ref, out_ref, acc_ref):
    i = pl.program_id(0)

    @pl.when(i == 0)
    def _init():
        acc_ref[0, 0] = 0.0

    diff = q_ref[...] - h_ref[...]
    acc_ref[0, 0] += jnp.sum(diff * diff)

    @pl.when(i == _NBL - 1)
    def _emit():
        mean = acc_ref[0, 0] / (_B * _D)
        out_ref[0, 0] = mean + _BETA * mean


_loss_call = pl.pallas_call(
    _loss_body,
    grid=(_NBL,),
    in_specs=[
        pl.BlockSpec((_BL, _D), lambda i: (i, 0)),
        pl.BlockSpec((_BL, _D), lambda i: (i, 0)),
    ],
    out_specs=pl.BlockSpec(memory_space=pltpu.SMEM),
    out_shape=jax.ShapeDtypeStruct((1, 1), jnp.float32),
    scratch_shapes=[pltpu.SMEM((1, 1), jnp.float32)],
)


def kernel(h, projections, codebooks):
    cc = _cnorm_call(codebooks)
    idx3 = _argmin_call(h, projections, codebooks, cc)
    offs = (jnp.arange(_G, dtype=jnp.int32) * _K)[:, None]
    flat_idx = (idx3[:, 0, :] + offs).T.reshape(-1)
    table = codebooks.reshape(_G * _K, _DG)
    quant = _gather_call()(table, flat_idx).reshape(_B, _D)
    loss = _loss_call(quant, h)[0, 0]
    return quant, loss


# speedup vs baseline: 3.1682x; 1.1030x over previous
import functools

import jax
import jax.numpy as jnp
from jax import lax
from jax.experimental import pallas as pl
from jax.experimental.pallas import tpu as pltpu
from jax.experimental.pallas import tpu_sc as plsc

_D = 2048
_G = 8
_K = 8192
_DG = _D // _G
_BETA = 0.25
_B = 4096

_BM = 1024
_KT = 4096
_NB = _B // _BM
_NK = _K // _KT
_HALF = _NK // 2
_JG = _KT // 128


def _cnorm_body(c_ref, out_ref):
    c = c_ref[0]
    out_ref[0, 0, :] = jnp.sum(c * c, axis=1)


_cnorm_call = pl.pallas_call(
    _cnorm_body,
    grid=(_G,),
    in_specs=[pl.BlockSpec((1, _K, _DG), lambda g: (g, 0, 0))],
    out_specs=pl.BlockSpec((1, 1, _K), lambda g: (g, 0, 0)),
    out_shape=jax.ShapeDtypeStruct((_G, 1, _K), jnp.float32),
)


def _argmin_body(h_ref, p_ref, c_ref, cc_ref, idx_ref,
                 z2_ref, zz_ref, lval_ref, lgrp_ref, best_ref, bidx_ref):
    i = pl.program_id(1)
    k = pl.program_id(2)

    @pl.when(k == 0)
    def _proj():
        z = lax.dot_general(h_ref[...], p_ref[0], (((1,), (1,)), ((), ())),
                            preferred_element_type=jnp.float32)
        z2_ref[...] = z + z
        zz_ref[...] = jnp.sum(z * z, axis=1, keepdims=True)

    @pl.when(k % _HALF == 0)
    def _chunk_init():
        lval_ref[...] = jnp.full((_BM, 128), jnp.inf, jnp.float32)
        lgrp_ref[...] = jnp.zeros((_BM, 128), jnp.int32)

    zc2 = lax.dot_general(z2_ref[...], c_ref[0], (((1,), (1,)), ((), ())),
                          preferred_element_type=jnp.float32)
    zz = zz_ref[...]
    cc = cc_ref[0, 0, :]

    lval = lval_ref[...]
    lgrp = lgrp_ref[...]
    for j in range(_JG):
        sl = (zz - zc2[:, j * 128:(j + 1) * 128]) + cc[j * 128:(j + 1) * 128]
        upd = sl < lval
        lgrp = jnp.where(upd, k * _JG + j, lgrp)
        lval = jnp.where(upd, sl, lval)
    lval_ref[...] = lval
    lgrp_ref[...] = lgrp

    lane = lax.broadcasted_iota(jnp.int32, (_BM, 128), 1)

    @pl.when(k == _HALF - 1)
    def _chunk0_done():
        v = lval_ref[...]
        m = jnp.min(v, axis=1, keepdims=True)
        cand = jnp.where(v == m, lgrp_ref[...] * 128 + lane, 2**31 - 1)
        bidx_ref[...] = jnp.min(cand, axis=1, keepdims=True)
        best_ref[...] = m.astype(jnp.bfloat16).astype(jnp.float32)

    @pl.when(k == _NK - 1)
    def _emit():
        v = lval_ref[...]
        m = jnp.min(v, axis=1, keepdims=True)
        cand = jnp.where(v == m, lgrp_ref[...] * 128 + lane, 2**31 - 1)
        ridx = jnp.min(cand, axis=1, keepdims=True)
        win = m < best_ref[...]
        idx_ref[0, 0, pl.ds(i * _BM, _BM)] = (
            jnp.where(win, ridx, bidx_ref[...]))[:, 0]


_argmin_call = pl.pallas_call(
    _argmin_body,
    grid=(_G, _NB, _NK),
    in_specs=[
        pl.BlockSpec((_BM, _D), lambda g, i, k: (i, 0)),
        pl.BlockSpec((1, _DG, _D), lambda g, i, k: (g, 0, 0)),
        pl.BlockSpec((1, _KT, _DG), lambda g, i, k: (g, k, 0)),
        pl.BlockSpec((1, 1, _KT), lambda g, i, k: (g, 0, k)),
    ],
    out_specs=pl.BlockSpec((1, 1, _B), lambda g, i, k: (g, 0, 0)),
    out_shape=jax.ShapeDtypeStruct((_G, 1, _B), jnp.int32),
    scratch_shapes=[
        pltpu.VMEM((_BM, _DG), jnp.float32),
        pltpu.VMEM((_BM, 1), jnp.float32),
        pltpu.VMEM((_BM, 128), jnp.float32),
        pltpu.VMEM((_BM, 128), jnp.int32),
        pltpu.VMEM((_BM, 1), jnp.float32),
        pltpu.VMEM((_BM, 1), jnp.int32),
    ],
)


_NC = 2
_NS = 16
_NW = _NC * _NS
_ROWS = _B * _G
_PER_W = _ROWS // _NW
_CH = 128
_NCH = _PER_W // _CH


def _gather_body(table_hbm, idx_hbm, out_hbm, idx_v, rows_v, sem):
    wid = lax.axis_index("s") * _NC + lax.axis_index("c")
    base = wid * _PER_W

    def _chunk(c, carry):
        off = base + c * _CH
        pltpu.sync_copy(idx_hbm.at[pl.ds(off, _CH)], idx_v)
        pltpu.async_copy(table_hbm.at[idx_v], rows_v, sem).wait()
        pltpu.sync_copy(rows_v, out_hbm.at[pl.ds(off, _CH)])
        return carry

    lax.fori_loop(0, _NCH, _chunk, 0)


@functools.cache
def _gather_call():
    return pl.kernel(
        _gather_body,
        out_type=jax.ShapeDtypeStruct((_ROWS, _DG), jnp.float32),
        mesh=plsc.VectorSubcoreMesh(core_axis_name="c", subcore_axis_name="s",
                                    num_cores=_NC, num_subcores=_NS),
        scratch_types=[
            pltpu.VMEM((_CH,), jnp.int32),
            pltpu.VMEM((_CH, _DG), jnp.float32),
            pltpu.SemaphoreType.DMA,
        ],
    )


_BL = 512
_NBL = _B // _BL


def _loss_body(q_ref, h_ref, out_ref, acc_ref):
    i = pl.program_id(0)

    @pl.when(i == 0)
    def _init():
        acc_ref[0, 0] = 0.0

    diff = q_ref[...] - h_ref[...]
    acc_ref[0, 0] += jnp.sum(diff * diff)

    @pl.when(i == _NBL - 1)
    def _emit():
        mean = acc_ref[0, 0] / (_B * _D)
        out_ref[0, 0] = mean + _BETA * mean


_loss_call = pl.pallas_call(
    _loss_body,
    grid=(_NBL,),
    in_specs=[
        pl.BlockSpec((_BL, _D), lambda i: (i, 0)),
        pl.BlockSpec((_BL, _D), lambda i: (i, 0)),
    ],
    out_specs=pl.BlockSpec(memory_space=pltpu.SMEM),
    out_shape=jax.ShapeDtypeStruct((1, 1), jnp.float32),
    scratch_shapes=[pltpu.SMEM((1, 1), jnp.float32)],
)


def kernel(h, projections, codebooks):
    cc = _cnorm_call(codebooks)
    idx3 = _argmin_call(h, projections, codebooks, cc)
    offs = (jnp.arange(_G, dtype=jnp.int32) * _K)[:, None]
    flat_idx = (idx3[:, 0, :] + offs).T.reshape(-1)
    table = codebooks.reshape(_G * _K, _DG)
    quant = _gather_call()(table, flat_idx).reshape(_B, _D)
    loss = _loss_call(quant, h)[0, 0]
    return quant, loss
